# Initial kernel scaffold; baseline (speedup 1.0000x reference)
import jax
import jax.numpy as jnp
from jax.experimental import pallas as pl


def kernel(input, embed, W, b):
    raise NotImplementedError("write your pallas kernel here")


# speedup vs baseline: 1.6600x; 1.6600x over previous
import functools

import jax
import jax.numpy as jnp
from jax import lax
from jax.experimental import pallas as pl
from jax.experimental.pallas import tpu as pltpu
from jax.experimental.pallas import tpu_sc as plsc

_NC = 2
_NS = 16
_LANES = 16


def _make_pool(B, L, E):
    NW = _NC * _NS
    BW = B // NW
    HL = L // 2
    CH = 64
    NSTEP = BW // CH
    EG = E // _LANES

    mesh = plsc.VectorSubcoreMesh(
        core_axis_name="c", subcore_axis_name="s",
        num_cores=_NC, num_subcores=_NS)

    @functools.partial(
        pl.kernel,
        mesh=mesh,
        compiler_params=pltpu.CompilerParams(use_tc_tiling_on_sc=False),
        out_type=jax.ShapeDtypeStruct((B, E), jnp.float32),
        scratch_types=[
            pltpu.VMEM((2 * CH, HL), jnp.int32),
            pltpu.VMEM((HL, E), jnp.float32),
            pltpu.VMEM((CH, E), jnp.float32),
            pltpu.SemaphoreType.DMA,
        ],
    )
    def pool(ids_hbm, tab_hbm, y_hbm, idx_v, rows_v, out_v, sem):
        wid = lax.axis_index("s") * _NC + lax.axis_index("c")
        for step in range(NSTEP):
            b0 = wid * BW + step * CH
            pltpu.sync_copy(ids_hbm.at[pl.ds(b0 * 2, 2 * CH)], idx_v)

            def batch_body(g, carry):
                accs = tuple(jnp.zeros((_LANES,), jnp.float32)
                             for _ in range(EG))
                for half in range(2):
                    pltpu.async_copy(
                        tab_hbm.at[idx_v.at[g * 2 + half]], rows_v, sem
                    ).wait()

                    def row_body(j, accs):
                        return tuple(
                            accs[c] + rows_v[j, pl.ds(_LANES * c, _LANES)]
                            for c in range(EG))

                    accs = lax.fori_loop(0, HL, row_body, accs)
                for c in range(EG):
                    out_v[g, pl.ds(_LANES * c, _LANES)] = accs[c]
                return carry

            lax.fori_loop(0, CH, batch_body, 0)
            pltpu.sync_copy(out_v, y_hbm.at[pl.ds(b0, CH)])

    return pool


def _mm_body(y_ref, wt_ref, b_ref, o_ref):
    o_ref[...] = jnp.dot(
        y_ref[...], wt_ref[...],
        preferred_element_type=jnp.float32,
        precision=lax.Precision.HIGHEST,
    ) + b_ref[...]


def _matmul(y, Wt, b2):
    B, E = y.shape
    N = Wt.shape[1]
    BB = 1024
    return pl.pallas_call(
        _mm_body,
        grid=(B // BB,),
        in_specs=[
            pl.BlockSpec((BB, E), lambda i: (i, 0)),
            pl.BlockSpec((E, N), lambda i: (0, 0)),
            pl.BlockSpec((1, N), lambda i: (0, 0)),
        ],
        out_specs=pl.BlockSpec((BB, N), lambda i: (i, 0)),
        out_shape=jax.ShapeDtypeStruct((B, N), jnp.float32),
    )(y, Wt, b2)


def kernel(input, embed, W, b):
    B, L = input.shape
    E = embed.shape[1]
    ids2 = input.reshape(2 * B, L // 2).astype(jnp.int32)
    y = _make_pool(B, L, E)(ids2, embed)
    return _matmul(y, W.T, b.reshape(1, -1))


# speedup vs baseline: 3.1270x; 1.8837x over previous
import functools

import jax
import jax.numpy as jnp
from jax import lax
from jax.experimental import pallas as pl
from jax.experimental.pallas import tpu as pltpu
from jax.experimental.pallas import tpu_sc as plsc

_NC = 2
_NS = 16
_LANES = 16


def _make_pool(B, L, E):
    NW = _NC * _NS
    BW = B // NW
    HL = L // 2
    CH = 64
    NSTEP = BW // CH
    EG = E // _LANES
    NBUF = 4
    PAIRS = CH // 2
    UNROLL = 5

    mesh = plsc.VectorSubcoreMesh(
        core_axis_name="c", subcore_axis_name="s",
        num_cores=_NC, num_subcores=_NS)

    @functools.partial(
        pl.kernel,
        mesh=mesh,
        compiler_params=pltpu.CompilerParams(use_tc_tiling_on_sc=False),
        out_type=jax.ShapeDtypeStruct((B, E), jnp.float32),
        scratch_types=[
            pltpu.VMEM((2 * CH, HL), jnp.int32),
            pltpu.VMEM((HL, E), jnp.float32),
            pltpu.VMEM((HL, E), jnp.float32),
            pltpu.VMEM((HL, E), jnp.float32),
            pltpu.VMEM((HL, E), jnp.float32),
            pltpu.VMEM((CH, E), jnp.float32),
            pltpu.SemaphoreType.DMA,
            pltpu.SemaphoreType.DMA,
            pltpu.SemaphoreType.DMA,
            pltpu.SemaphoreType.DMA,
        ],
    )
    def pool(ids_hbm, tab_hbm, y_hbm, idx_v, r0, r1, r2, r3, out_v,
             s0, s1, s2, s3):
        bufs = (r0, r1, r2, r3)
        sems = (s0, s1, s2, s3)
        wid = lax.axis_index("s") * _NC + lax.axis_index("c")

        def accumulate(rows, accs):
            def row_body(j, accs):
                a = list(accs)
                for u in range(UNROLL):
                    r = j * UNROLL + u
                    for c in range(EG):
                        a[c] = a[c] + rows[r, pl.ds(_LANES * c, _LANES)]
                return tuple(a)
            return lax.fori_loop(0, HL // UNROLL, row_body, accs)

        for step in range(NSTEP):
            b0 = wid * BW + step * CH
            pltpu.sync_copy(ids_hbm.at[pl.ds(b0 * 2, 2 * CH)], idx_v)
            for q in range(NBUF):
                pltpu.async_copy(tab_hbm.at[idx_v.at[q]], bufs[q], sems[q])

            def pair_body(p, carry):
                for pair in range(2):
                    accs = tuple(jnp.zeros((_LANES,), jnp.float32)
                                 for _ in range(EG))
                    for half in range(2):
                        q = 2 * pair + half
                        h = 4 * p + q
                        pltpu.make_async_copy(
                            tab_hbm.at[idx_v.at[h]], bufs[q], sems[q]
                        ).wait()
                        accs = accumulate(bufs[q], accs)

                        @pl.when(p < PAIRS - 1)
                        def _():
                            pltpu.async_copy(
                                tab_hbm.at[idx_v.at[h + 4]], bufs[q], sems[q])
                    for c in range(EG):
                        out_v[2 * p + pair, pl.ds(_LANES * c, _LANES)] = accs[c]
                return carry

            lax.fori_loop(0, PAIRS, pair_body, 0)
            pltpu.sync_copy(out_v, y_hbm.at[pl.ds(b0, CH)])

    return pool


def _mm_body(y_ref, wt_ref, b_ref, o_ref):
    o_ref[...] = jnp.dot(
        y_ref[...], wt_ref[...],
        preferred_element_type=jnp.float32,
        precision=lax.Precision.HIGHEST,
    ) + b_ref[...]


def _matmul(y, Wt, b2):
    B, E = y.shape
    N = Wt.shape[1]
    BB = 1024
    return pl.pallas_call(
        _mm_body,
        grid=(B // BB,),
        in_specs=[
            pl.BlockSpec((BB, E), lambda i: (i, 0)),
            pl.BlockSpec((E, N), lambda i: (0, 0)),
            pl.BlockSpec((1, N), lambda i: (0, 0)),
        ],
        out_specs=pl.BlockSpec((BB, N), lambda i: (i, 0)),
        out_shape=jax.ShapeDtypeStruct((B, N), jnp.float32),
    )(y, Wt, b2)


def kernel(input, embed, W, b):
    B, L = input.shape
    E = embed.shape[1]
    ids2 = input.reshape(2 * B, L // 2).astype(jnp.int32)
    y = _make_pool(B, L, E)(ids2, embed)
    return _matmul(y, W.T, b.reshape(1, -1))


# speedup vs baseline: 4.0051x; 1.2808x over previous
import functools

import jax
import jax.numpy as jnp
from jax import lax
from jax.experimental import pallas as pl
from jax.experimental.pallas import tpu as pltpu
from jax.experimental.pallas import tpu_sc as plsc

_NC = 2
_NS = 16
_LANES = 16


def _make_pool(B, L, E):
    NW = _NC * _NS
    BW = B // NW
    HL = L // 2
    CH = 64
    NSTEP = BW // CH
    EG = E // _LANES
    NBUF = 4
    PAIRS = CH // 2
    UNROLL = 5

    mesh = plsc.VectorSubcoreMesh(
        core_axis_name="c", subcore_axis_name="s",
        num_cores=_NC, num_subcores=_NS)

    @functools.partial(
        pl.kernel,
        mesh=mesh,
        compiler_params=pltpu.CompilerParams(use_tc_tiling_on_sc=False),
        out_type=jax.ShapeDtypeStruct((B, E), jnp.float32),
        scratch_types=[
            pltpu.VMEM((2 * CH, HL), jnp.int32),
            pltpu.VMEM((HL, E), jnp.float32),
            pltpu.VMEM((HL, E), jnp.float32),
            pltpu.VMEM((HL, E), jnp.float32),
            pltpu.VMEM((HL, E), jnp.float32),
            pltpu.VMEM((CH, E), jnp.float32),
            pltpu.SemaphoreType.DMA,
            pltpu.SemaphoreType.DMA,
            pltpu.SemaphoreType.DMA,
            pltpu.SemaphoreType.DMA,
        ],
    )
    def pool(ids_hbm, tab_hbm, y_hbm, idx_v, r0, r1, r2, r3, out_v,
             s0, s1, s2, s3):
        bufs = (r0, r1, r2, r3)
        sems = (s0, s1, s2, s3)
        wid = lax.axis_index("s") * _NC + lax.axis_index("c")

        def accumulate(rows, accs):
            def row_body(j, accs):
                a = list(accs)
                for u in range(UNROLL):
                    r = j * UNROLL + u
                    for c in range(EG):
                        a[c] = a[c] + rows[r, pl.ds(_LANES * c, _LANES)]
                return tuple(a)
            return lax.fori_loop(0, HL // UNROLL, row_body, accs)

        for step in range(NSTEP):
            b0 = wid * BW + step * CH
            pltpu.sync_copy(ids_hbm.at[pl.ds(b0 * 2, 2 * CH)], idx_v)
            for q in range(NBUF):
                pltpu.async_copy(tab_hbm.at[idx_v.at[q]], bufs[q], sems[q])

            def pair_body(p, carry):
                for pair in range(2):
                    accs = tuple(jnp.zeros((_LANES,), jnp.float32)
                                 for _ in range(EG))
                    for half in range(2):
                        q = 2 * pair + half
                        h = 4 * p + q
                        pltpu.make_async_copy(
                            tab_hbm.at[idx_v.at[h]], bufs[q], sems[q]
                        ).wait()
                        accs = accumulate(bufs[q], accs)

                        @pl.when(p < PAIRS - 1)
                        def _():
                            pltpu.async_copy(
                                tab_hbm.at[idx_v.at[h + 4]], bufs[q], sems[q])
                    for c in range(EG):
                        out_v[2 * p + pair, pl.ds(_LANES * c, _LANES)] = accs[c]
                return carry

            lax.fori_loop(0, PAIRS, pair_body, 0)
            pltpu.sync_copy(out_v, y_hbm.at[pl.ds(b0, CH)])

    return pool


def _tr_body(x_ref, o_ref):
    o_ref[:, 0:64] = x_ref[...].T


def _transpose_pad(embT):
    E, V = embT.shape
    TB = 4096
    grid = (V + TB - 1) // TB
    return pl.pallas_call(
        _tr_body,
        grid=(grid,),
        in_specs=[pl.BlockSpec((E, TB), lambda i: (0, i))],
        out_specs=pl.BlockSpec((TB, 2 * E), lambda i: (i, 0)),
        out_shape=jax.ShapeDtypeStruct((V, 2 * E), jnp.float32),
    )(embT)


def _mm_body(y_ref, wt_ref, b_ref, o_ref):
    o_ref[...] = jnp.dot(
        y_ref[...], wt_ref[...],
        preferred_element_type=jnp.float32,
        precision=lax.Precision.HIGHEST,
    ) + b_ref[...]


def _matmul(y, Wt, b2):
    B, E = y.shape
    N = Wt.shape[1]
    BB = 1024
    return pl.pallas_call(
        _mm_body,
        grid=(B // BB,),
        in_specs=[
            pl.BlockSpec((BB, E), lambda i: (i, 0)),
            pl.BlockSpec((E, N), lambda i: (0, 0)),
            pl.BlockSpec((1, N), lambda i: (0, 0)),
        ],
        out_specs=pl.BlockSpec((BB, N), lambda i: (i, 0)),
        out_shape=jax.ShapeDtypeStruct((B, N), jnp.float32),
    )(y, Wt, b2)


def kernel(input, embed, W, b):
    B, L = input.shape
    V, E = embed.shape
    table2 = _transpose_pad(embed.T).reshape(2 * V, E)
    ids2 = (input.astype(jnp.int32) * 2).reshape(2 * B, L // 2)
    y = _make_pool(B, L, E)(ids2, table2)
    return _matmul(y, W.T, b.reshape(1, -1))
